# baseline (speedup 1.0000x reference)
import jax
import jax.numpy as jnp
from jax.experimental import pallas as pl
from jax.experimental.pallas import tpu as pltpu

_FRAC = 0.5


def _copy_block(src_ref, x_ref, o_ref):
    del src_ref
    o_ref[...] = x_ref[...]


def kernel(X):
    B, C, H, W = X.shape
    num = int(B * _FRAC)
    num = num if not num % 2 else num - 1
    num = max(2, num)
    half = num // 2

    ch_key = jax.random.key(42)
    channel = jax.random.randint(ch_key, (half,), 0, C)

    src_b = jnp.broadcast_to(jnp.arange(B, dtype=jnp.int32)[:, None], (B, C))
    i = jnp.arange(half, dtype=jnp.int32)
    src_b = src_b.at[i, channel].set(i + half)
    src_b = src_b.at[i + half, channel].set(i)

    out = pl.pallas_call(
        _copy_block,
        grid_spec=pltpu.PrefetchScalarGridSpec(
            num_scalar_prefetch=1,
            grid=(B, C),
            in_specs=[
                pl.BlockSpec((1, 1, H, W), lambda b, c, s: (s[b, c], c, 0, 0)),
            ],
            out_specs=pl.BlockSpec((1, 1, H, W), lambda b, c, s: (b, c, 0, 0)),
        ),
        out_shape=jax.ShapeDtypeStruct(X.shape, X.dtype),
    )(src_b, X)

    return (out, jnp.arange(num))


# speedup vs baseline: 4.0421x; 4.0421x over previous
import jax
import jax.numpy as jnp
from jax.experimental import pallas as pl
from jax.experimental.pallas import tpu as pltpu

_FRAC = 0.5
_COPY_BLOCK = 32


def _copy_block(x_ref, o_ref):
    o_ref[...] = x_ref[...]


def _swap_block(s_ref, x_ref, b_ref, o_ref):
    del s_ref, b_ref
    o_ref[...] = x_ref[...]


def kernel(X):
    B, C, H, W = X.shape
    num = int(B * _FRAC)
    num = num if not num % 2 else num - 1
    num = max(2, num)
    half = num // 2

    ch_key = jax.random.key(42)
    channel = jax.random.randint(ch_key, (half,), 0, C)

    Xf = X.reshape(B * C, H, W)

    buf = pl.pallas_call(
        _copy_block,
        grid=(B * C // _COPY_BLOCK,),
        in_specs=[pl.BlockSpec((_COPY_BLOCK, H, W), lambda i: (i, 0, 0))],
        out_specs=pl.BlockSpec((_COPY_BLOCK, H, W), lambda i: (i, 0, 0)),
        out_shape=jax.ShapeDtypeStruct(Xf.shape, Xf.dtype),
    )(Xf)

    i = jnp.arange(num, dtype=jnp.int32)
    ch2 = jnp.concatenate([channel, channel]).astype(jnp.int32)
    dst = i * C + ch2
    src = ((i + half) % num) * C + ch2
    swaps = jnp.stack([src, dst])

    out = pl.pallas_call(
        _swap_block,
        grid_spec=pltpu.PrefetchScalarGridSpec(
            num_scalar_prefetch=1,
            grid=(num,),
            in_specs=[
                pl.BlockSpec((1, H, W), lambda i, s: (s[0, i], 0, 0)),
                pl.BlockSpec((1, H, W), lambda i, s: (s[1, i], 0, 0)),
            ],
            out_specs=pl.BlockSpec((1, H, W), lambda i, s: (s[1, i], 0, 0)),
        ),
        out_shape=jax.ShapeDtypeStruct(Xf.shape, Xf.dtype),
        input_output_aliases={2: 0},
    )(swaps, Xf, buf)

    return (out.reshape(B, C, H, W), jnp.arange(num))
